# baseline (speedup 1.0000x reference)
import functools

import jax
import jax.numpy as jnp
from jax import lax
from jax.experimental import pallas as pl
from jax.experimental.pallas import tpu as pltpu
from jax.experimental.pallas import tpu_sc as plsc

_L = 16
_NW = 32
_CHUNK = 16384


def _sc_body(x_hbm, pos_hbm, scale_hbm, out_hbm, xbuf, pbuf, sbuf):
    c = lax.axis_index("c")
    s = lax.axis_index("s")
    wid = s * 2 + c
    n_per_w = x_hbm.shape[0] // _NW
    x0 = wid * n_per_w
    p0 = lax.rem(x0, pos_hbm.shape[0])
    pltpu.sync_copy(scale_hbm, sbuf)
    sv = sbuf[...]

    def chunk_body(j, carry):
        off = j * _CHUNK
        pltpu.sync_copy(x_hbm.at[pl.ds(x0 + off, _CHUNK)], xbuf)
        pltpu.sync_copy(pos_hbm.at[pl.ds(p0 + off, _CHUNK)], pbuf)

        def vec_body(i, carry2):
            ix = pl.ds(i * _L, _L)
            xbuf[ix] = xbuf[ix] + sv * pbuf[ix]
            return carry2

        lax.fori_loop(0, _CHUNK // _L, vec_body, 0)
        pltpu.sync_copy(xbuf, out_hbm.at[pl.ds(x0 + off, _CHUNK)])
        return carry

    lax.fori_loop(0, n_per_w // _CHUNK, chunk_body, 0)


def kernel(x, pos_table, use_pos_embed):
    batch, seq_len, embed_dim = x.shape
    scale = jnp.asarray(use_pos_embed, jnp.float32)
    scale16 = jnp.full((_L,), scale, jnp.float32)
    x_flat = x.reshape(-1)
    pos_flat = pos_table[:seq_len].reshape(-1)
    mesh = plsc.VectorSubcoreMesh(core_axis_name="c", subcore_axis_name="s")
    sc_add = functools.partial(
        pl.kernel,
        out_type=jax.ShapeDtypeStruct(x_flat.shape, x_flat.dtype),
        mesh=mesh,
        scratch_types=[
            pltpu.VMEM((_CHUNK,), jnp.float32),
            pltpu.VMEM((_CHUNK,), jnp.float32),
            pltpu.VMEM((_L,), jnp.float32),
        ],
    )(_sc_body)
    out_flat = sc_add(x_flat, pos_flat, scale16)
    return out_flat.reshape(x.shape)


# speedup vs baseline: 1.4566x; 1.4566x over previous
import functools

import jax
import jax.numpy as jnp
from jax import lax
from jax.experimental import pallas as pl
from jax.experimental.pallas import tpu as pltpu
from jax.experimental.pallas import tpu_sc as plsc

_L = 16
_NW = 32
_CHUNK = 16384


def _sc_body(x_hbm, pos_hbm, scale_hbm, out_hbm, xbuf, pbuf, sbuf):
    c = lax.axis_index("c")
    s = lax.axis_index("s")
    wid = s * 2 + c
    n_per_w = x_hbm.shape[0] // _NW
    x0 = wid * n_per_w
    p0 = lax.rem(x0, pos_hbm.shape[0])
    pltpu.sync_copy(scale_hbm, sbuf)
    sv = sbuf[...]

    def chunk_body(j, carry):
        off = j * _CHUNK
        pltpu.sync_copy(x_hbm.at[pl.ds(x0 + off, _CHUNK)], xbuf)
        pltpu.sync_copy(pos_hbm.at[pl.ds(p0 + off, _CHUNK)], pbuf)

        @plsc.parallel_loop(0, _CHUNK, _L, unroll=8)
        def vec_body(i):
            ix = pl.ds(i, _L)
            xbuf[ix] = xbuf[ix] + sv * pbuf[ix]

        pltpu.sync_copy(xbuf, out_hbm.at[pl.ds(x0 + off, _CHUNK)])
        return carry

    lax.fori_loop(0, n_per_w // _CHUNK, chunk_body, 0)


def kernel(x, pos_table, use_pos_embed):
    batch, seq_len, embed_dim = x.shape
    scale = jnp.asarray(use_pos_embed, jnp.float32)
    scale16 = jnp.full((_L,), scale, jnp.float32)
    x_flat = x.reshape(-1)
    pos_flat = pos_table[:seq_len].reshape(-1)
    mesh = plsc.VectorSubcoreMesh(core_axis_name="c", subcore_axis_name="s")
    sc_add = functools.partial(
        pl.kernel,
        out_type=jax.ShapeDtypeStruct(x_flat.shape, x_flat.dtype),
        mesh=mesh,
        scratch_types=[
            pltpu.VMEM((_CHUNK,), jnp.float32),
            pltpu.VMEM((_CHUNK,), jnp.float32),
            pltpu.VMEM((_L,), jnp.float32),
        ],
    )(_sc_body)
    out_flat = sc_add(x_flat, pos_flat, scale16)
    return out_flat.reshape(x.shape)


# speedup vs baseline: 2.6541x; 1.8220x over previous
import functools

import jax
import jax.numpy as jnp
from jax import lax
from jax.experimental import pallas as pl
from jax.experimental.pallas import tpu as pltpu
from jax.experimental.pallas import tpu_sc as plsc

_L = 16
_NW = 32
_CHUNK = 16384
_S_BLK = 2048


def _sc_body(x_hbm, pos_hbm, scale_hbm, out_hbm, xbuf, pbuf, sbuf):
    c = lax.axis_index("c")
    s = lax.axis_index("s")
    wid = s * 2 + c
    n_per_w = out_hbm.shape[0] // _NW
    x0 = wid * n_per_w
    pltpu.sync_copy(scale_hbm, sbuf)
    sv = sbuf[...]

    def chunk_body(j, carry):
        off = x0 + j * _CHUNK
        pltpu.sync_copy(x_hbm.at[pl.ds(off, _CHUNK)], xbuf)
        pltpu.sync_copy(pos_hbm.at[pl.ds(off, _CHUNK)], pbuf)

        @plsc.parallel_loop(0, _CHUNK, _L, unroll=8)
        def vec_body(i):
            ix = pl.ds(i, _L)
            xbuf[ix] = xbuf[ix] + sv * pbuf[ix]

        pltpu.sync_copy(xbuf, out_hbm.at[pl.ds(off, _CHUNK)])
        return carry

    lax.fori_loop(0, n_per_w // _CHUNK, chunk_body, 0)


def _add_body(scale_ref, x_ref, pos_ref, o_ref):
    o_ref[...] = x_ref[...] + scale_ref[0] * pos_ref[...]


def kernel(x, pos_table, use_pos_embed):
    batch, seq_len, embed_dim = x.shape
    scale = jnp.asarray(use_pos_embed, jnp.float32)

    sc_n = seq_len * embed_dim
    mesh = plsc.VectorSubcoreMesh(core_axis_name="c", subcore_axis_name="s")
    sc_add = functools.partial(
        pl.kernel,
        out_type=jax.ShapeDtypeStruct((sc_n,), x.dtype),
        mesh=mesh,
        scratch_types=[
            pltpu.VMEM((_CHUNK,), jnp.float32),
            pltpu.VMEM((_CHUNK,), jnp.float32),
            pltpu.VMEM((_L,), jnp.float32),
        ],
    )(_sc_body)
    sc_out = sc_add(
        x.reshape(-1),
        pos_table[:seq_len].reshape(-1),
        jnp.full((_L,), scale, jnp.float32),
    )

    grid = (seq_len // _S_BLK, batch - 1)
    tc_full = pl.pallas_call(
        _add_body,
        grid=grid,
        in_specs=[
            pl.BlockSpec(memory_space=pltpu.SMEM),
            pl.BlockSpec((1, _S_BLK, embed_dim), lambda i, b: (b + 1, i, 0)),
            pl.BlockSpec((_S_BLK, embed_dim), lambda i, b: (i, 0)),
        ],
        out_specs=pl.BlockSpec((1, _S_BLK, embed_dim), lambda i, b: (b + 1, i, 0)),
        out_shape=jax.ShapeDtypeStruct(x.shape, x.dtype),
    )(scale.reshape((1,)), x, pos_table[:seq_len])

    return lax.dynamic_update_slice(
        tc_full, sc_out.reshape(1, seq_len, embed_dim), (0, 0, 0)
    )


# speedup vs baseline: 8.9409x; 3.3688x over previous
import jax
import jax.numpy as jnp
from jax import lax
from jax.experimental import pallas as pl
from jax.experimental.pallas import tpu as pltpu

_R_BLK = 2048


def _add_body(scale_ref, x_ref, pos_ref, o_ref):
    o_ref[...] = x_ref[...] + scale_ref[0] * pos_ref[...]


def kernel(x, pos_table, use_pos_embed):
    batch, seq_len, embed_dim = x.shape
    scale = jnp.asarray(use_pos_embed, jnp.float32).reshape((1,))
    x2d = x.reshape(batch * seq_len, embed_dim)
    period = seq_len // _R_BLK
    grid = (batch * seq_len // _R_BLK,)
    out = pl.pallas_call(
        _add_body,
        grid=grid,
        in_specs=[
            pl.BlockSpec(memory_space=pltpu.SMEM),
            pl.BlockSpec(
                (_R_BLK, embed_dim),
                lambda i: (lax.rem(i, batch) * period + lax.div(i, batch), 0),
            ),
            pl.BlockSpec((_R_BLK, embed_dim), lambda i: (lax.div(i, batch), 0)),
        ],
        out_specs=pl.BlockSpec(
            (_R_BLK, embed_dim),
            lambda i: (lax.rem(i, batch) * period + lax.div(i, batch), 0),
        ),
        out_shape=jax.ShapeDtypeStruct(x2d.shape, x.dtype),
    )(scale, x2d, pos_table[:seq_len])
    return out.reshape(x.shape)


# speedup vs baseline: 8.9480x; 1.0008x over previous
import jax
import jax.numpy as jnp
from jax import lax
from jax.experimental import pallas as pl
from jax.experimental.pallas import tpu as pltpu

_R_BLK = 2048


def _add_body(scale_ref, x_ref, pos_ref, o_ref):
    o_ref[...] = x_ref[...] + scale_ref[0] * pos_ref[...]


def kernel(x, pos_table, use_pos_embed):
    batch, seq_len, embed_dim = x.shape
    scale = jnp.asarray(use_pos_embed, jnp.float32).reshape((1,))
    x2d = x.reshape(batch * seq_len, embed_dim)
    period = seq_len // _R_BLK
    grid = (batch * seq_len // _R_BLK,)
    out = pl.pallas_call(
        _add_body,
        grid=grid,
        in_specs=[
            pl.BlockSpec(memory_space=pltpu.SMEM),
            pl.BlockSpec(
                (_R_BLK, embed_dim),
                lambda i: (lax.rem(i, batch) * period + lax.div(i, batch), 0),
            ),
            pl.BlockSpec((_R_BLK, embed_dim), lambda i: (lax.div(i, batch), 0)),
        ],
        out_specs=pl.BlockSpec(
            (_R_BLK, embed_dim),
            lambda i: (lax.rem(i, batch) * period + lax.div(i, batch), 0),
        ),
        out_shape=jax.ShapeDtypeStruct(x2d.shape, x.dtype),
        compiler_params=pltpu.CompilerParams(
            dimension_semantics=("arbitrary",),
        ),
    )(scale, x2d, pos_table[:seq_len])
    return out.reshape(x.shape)
